# baseline (speedup 1.0000x reference)
import jax
import jax.numpy as jnp
from jax import lax
from jax.experimental import pallas as pl
from jax.experimental.pallas import tpu as pltpu
from jax.experimental.pallas import tpu_sc as plsc

_B = 2
_H = 128
_W = 128
_L = 4
_D = 256
_NQ = 600
_NQH = _NQ // 2
_NSUB = 16
_QPS = (_NQH + _NSUB - 1) // _NSUB
_SLOTS = 20
_HW = _H * _W
_NFLAT = _B * _HW * _NQH
_ZPS = _NFLAT // _B // _NSUB
_ZBUF = 38400
_NZD = _ZPS // _ZBUF


def _slice_body(in_ref, oe_ref, oo_ref):
    x = in_ref[0, 0]
    oe_ref[...] = x[:, :128]
    oo_ref[...] = x[:, 128:]


def _mlp_body(q_ref, w0, b0, w1, b1, w2, b2, w3, b3, o_ref):
    x = q_ref[...]
    for w_r, b_r in ((w0, b0), (w1, b1), (w2, b2)):
        x = lax.dot_general(x, w_r[...], (((1,), (1,)), ((), ())),
                            preferred_element_type=jnp.float32) + b_r[...]
        x = jnp.maximum(x, 0.0)
    x = lax.dot_general(x, w3[...], (((1,), (1,)), ((), ())),
                        preferred_element_type=jnp.float32) + b3[...]
    o_ref[...] = x


def _sc_body(feat_e, feat_o, qe, pos, out,
             zbuf, posv, qeidx, qeown, qmeti, gidxbuf, wts, rows_e, rows_o,
             logbuf, sibuf, red,
             g0, g1, g2, g3, zsem, ssem):
    img = lax.axis_index("c")
    s = lax.axis_index("s")
    gsems = (g0, g1, g2, g3)
    iota = lax.iota(jnp.int32, 16)
    zeros16 = jnp.zeros((16,), jnp.float32)

    def _zb(i, carry):
        for u in range(8):
            zbuf[pl.ds(i * 128 + u * 16, 16)] = zeros16
        return carry

    lax.fori_loop(0, _ZBUF // 128, _zb, 0)
    zbase = img * (_NFLAT // _B) + s * _ZPS

    pltpu.sync_copy(pos, posv.at[pl.ds(0, _NQ * 2)])
    count = jnp.minimum(_QPS, _NQH - s * _QPS)
    qid0 = img * _NQH + s * _QPS
    for m in range(2):
        qloc = iota + (16 * m)
        qcap = jnp.minimum(qloc, count - 1)
        qeidx[pl.ds(16 * m, 16)] = qid0 + qcap
    qcp = pltpu.async_copy(qe.at[qeidx], qeown, ssem)
    for m in range(3):
        v = posv[pl.ds(qid0 * 2 + 16 * m, 16)]
        qmeti[pl.ds(16 * m, 16)] = (v * 128.0).astype(jnp.int32)
    qcp.wait()

    pixbase = img * _HW

    def _build(slot, b):
        qd = jnp.minimum(slot, count - 1)
        pvi = qmeti[pl.ds(qd * 2, 16)]
        cc = pvi[0]
        rc = pvi[1]
        r0 = jnp.maximum(rc - 3, 0)
        r1 = jnp.minimum(rc + 3, _H - 1)
        c0 = jnp.maximum(cc - 3, 0)
        c1 = jnp.minimum(cc + 3, _W - 1)
        ch = s * _QPS + qd
        pair = lax.shift_right_logical(slot, 1)
        par = jnp.bitwise_and(slot, 1)
        for m in range(4):
            k = iota + 16 * m
            a = lax.shift_right_logical(k * 9363, 16)
            kb = k - a * 7
            h = jnp.minimum(r0 + a, r1)
            w = jnp.minimum(c0 + kb, c1)
            mr = jnp.where(h == 0, 4 - rc,
                           jnp.where(h == _H - 1, rc - (_H - 5), 1))
            mc = jnp.where(w == 0, 4 - cc,
                           jnp.where(w == _W - 1, cc - (_W - 5), 1))
            pix = pixbase + h * _W + w
            gidxbuf[b, pl.ds(16 * m, 16)] = pix
            sibuf[pair, pl.ds(par * 64 + 16 * m, 16)] = pix * _NQH + ch
            wts[b, pl.ds(16 * m, 16)] = (mr * mc).astype(jnp.float32)

    def _issue(b):
        pltpu.async_copy(feat_e.at[gidxbuf.at[b]], rows_e.at[b], gsems[b])
        pltpu.async_copy(feat_o.at[gidxbuf.at[b]], rows_o.at[b], gsems[b])

    def _compute(slot, b):
        qd = jnp.minimum(slot, count - 1)
        qv = [qeown[qd, pl.ds(16 * t, 16)] for t in range(16)]
        for m in range(4):
            def _cell(ci, lv):
                cell = 16 * m + ci
                accs = [rows_e[b, cell, pl.ds(16 * t, 16)] * qv[t]
                        for t in range(4)]
                for t in range(4, 8):
                    accs[t % 4] = (accs[t % 4]
                                   + rows_e[b, cell, pl.ds(16 * t, 16)] * qv[t])
                for t in range(8, 16):
                    accs[t % 4] = (accs[t % 4]
                                   + rows_o[b, cell, pl.ds(16 * (t - 8), 16)]
                                   * qv[t])
                acc = (accs[0] + accs[1]) + (accs[2] + accs[3])
                for shift in (8, 4, 2, 1):
                    red[pl.ds(0, 16)] = acc
                    red[pl.ds(16, 16)] = acc
                    acc = acc + red[pl.ds(shift, 16)]
                return jnp.where(iota == ci, acc, lv)

            lv = lax.fori_loop(0, 16, _cell, jnp.zeros((16,), jnp.float32))
            logbuf[pl.ds(slot * 64 + 16 * m, 16)] = (
                lv * wts[b, pl.ds(16 * m, 16)])

    for b in range(4):
        _build(b, b)
        _issue(b)

    def _group(grp, carry):
        for b in range(4):
            slot = grp * 4 + b
            pltpu.make_async_copy(feat_e.at[gidxbuf.at[b]], rows_e.at[b],
                                  gsems[b]).wait()
            pltpu.make_async_copy(feat_o.at[gidxbuf.at[b]], rows_o.at[b],
                                  gsems[b]).wait()
            _compute(slot, b)

            @pl.when(grp < 4)
            def _():
                _build(slot + 4, b)
                _issue(b)

        @pl.when(grp < 4)
        def _():
            for u in range(2):
                pltpu.async_copy(
                    zbuf,
                    out.at[pl.ds(zbase + (grp * 2 + u) * _ZBUF, _ZBUF)],
                    zsem)
        return carry

    lax.fori_loop(0, 5, _group, 0)

    for j in range(_NZD):
        pltpu.make_async_copy(
            zbuf, out.at[pl.ds(zbase + j * _ZBUF, _ZBUF)], zsem).wait()
    plsc.subcore_barrier()
    sds = [pltpu.async_copy(logbuf.at[pl.ds(128 * j, 128)],
                            out.at[sibuf.at[j]], ssem)
           for j in range(_SLOTS // 2)]
    for d in sds:
        d.wait()


def kernel(stacked_feature_map, queries, query_batch_offsets,
           query_positions, image_spatial_shapes,
           W0, b0, W1, b1, W2, b2, W3, b3):
    qe = pl.pallas_call(
        _mlp_body,
        out_shape=jax.ShapeDtypeStruct((_NQ, _D), jnp.float32),
    )(queries, W0, b0.reshape(1, _D), W1, b1.reshape(1, _D),
      W2, b2.reshape(1, _D), W3, b3.reshape(1, _D))

    full = stacked_feature_map[..., _L - 1, :]
    feat_e, feat_o = pl.pallas_call(
        _slice_body,
        grid=(_B * _H,),
        in_specs=[pl.BlockSpec((1, 1, _W, _D),
                               lambda p: (p // _H, p % _H, 0, 0))],
        out_specs=[pl.BlockSpec((_W, 128), lambda p: (p, 0)),
                   pl.BlockSpec((_W, 128), lambda p: (p, 0))],
        out_shape=[jax.ShapeDtypeStruct((_B * _HW, 128), jnp.float32),
                   jax.ShapeDtypeStruct((_B * _HW, 128), jnp.float32)],
    )(full)
    posf = query_positions.reshape(_NQ * 2)

    sc_fn = pl.kernel(
        _sc_body,
        out_type=jax.ShapeDtypeStruct((_NFLAT,), jnp.float32),
        mesh=plsc.VectorSubcoreMesh(core_axis_name="c", subcore_axis_name="s"),
        scratch_types=[
            pltpu.VMEM((_ZBUF,), jnp.float32),
            pltpu.VMEM((_NQ * 2 + 32,), jnp.float32),
            pltpu.VMEM((32,), jnp.int32),
            pltpu.VMEM((32, _D), jnp.float32),
            pltpu.VMEM((64,), jnp.int32),
            pltpu.VMEM((4, 64), jnp.int32),
            pltpu.VMEM((4, 64), jnp.float32),
            pltpu.VMEM((4, 64, 128), jnp.float32),
            pltpu.VMEM((4, 64, 128), jnp.float32),
            pltpu.VMEM((_SLOTS * 64,), jnp.float32),
            pltpu.VMEM((_SLOTS // 2, 128), jnp.int32),
            pltpu.VMEM((32,), jnp.float32),
            pltpu.SemaphoreType.DMA,
            pltpu.SemaphoreType.DMA,
            pltpu.SemaphoreType.DMA,
            pltpu.SemaphoreType.DMA,
            pltpu.SemaphoreType.DMA,
            pltpu.SemaphoreType.DMA,
        ],
    )
    outf = sc_fn(feat_e, feat_o, qe, posf)
    return outf.reshape(_B, _H, _W, _NQH)


# speedup vs baseline: 1.4594x; 1.4594x over previous
import jax
import jax.numpy as jnp
from jax import lax
from jax.experimental import pallas as pl
from jax.experimental.pallas import tpu as pltpu
from jax.experimental.pallas import tpu_sc as plsc

_B = 2
_H = 128
_W = 128
_L = 4
_D = 256
_NQ = 600
_NQH = _NQ // 2
_NSUB = 16
_QPS = (_NQH + _NSUB - 1) // _NSUB
_SLOTS = 20
_HW = _H * _W
_NFLAT = _B * _HW * _NQH
_ZPS = _NFLAT // _B // _NSUB
_ZBUF = 38400
_NZD = _ZPS // _ZBUF


def _mlp_body(q_ref, w0, b0, w1, b1, w2, b2, w3, b3, o_ref):
    x = q_ref[...]
    for w_r, b_r in ((w0, b0), (w1, b1), (w2, b2)):
        x = lax.dot_general(x, w_r[...], (((1,), (1,)), ((), ())),
                            preferred_element_type=jnp.float32) + b_r[...]
        x = jnp.maximum(x, 0.0)
    x = lax.dot_general(x, w3[...], (((1,), (1,)), ((), ())),
                        preferred_element_type=jnp.float32) + b3[...]
    o_ref[...] = x


def _sc_body(feat, qe, pos, out,
             zbuf, posv, qeidx, qeown, qmeti, gidxbuf, wts, rows,
             logbuf, sibuf, red,
             g0, g1, g2, g3, zsem, ssem):
    img = lax.axis_index("c")
    s = lax.axis_index("s")
    gsems = (g0, g1, g2, g3)
    iota = lax.iota(jnp.int32, 16)
    zeros16 = jnp.zeros((16,), jnp.float32)

    def _zb(i, carry):
        for u in range(8):
            zbuf[pl.ds(i * 128 + u * 16, 16)] = zeros16
        return carry

    lax.fori_loop(0, _ZBUF // 128, _zb, 0)
    zbase = img * (_NFLAT // _B) + s * _ZPS

    pltpu.sync_copy(pos, posv.at[pl.ds(0, _NQ * 2)])
    count = jnp.minimum(_QPS, _NQH - s * _QPS)
    qid0 = img * _NQH + s * _QPS
    for m in range(2):
        qloc = iota + (16 * m)
        qcap = jnp.minimum(qloc, count - 1)
        qeidx[pl.ds(16 * m, 16)] = qid0 + qcap
    qcp = pltpu.async_copy(qe.at[qeidx], qeown, ssem)
    for m in range(3):
        v = posv[pl.ds(qid0 * 2 + 16 * m, 16)]
        qmeti[pl.ds(16 * m, 16)] = (v * 128.0).astype(jnp.int32)
    qcp.wait()

    pixbase = img * _HW

    def _build(slot, b):
        qd = jnp.minimum(slot, count - 1)
        pvi = qmeti[pl.ds(qd * 2, 16)]
        cc = pvi[0]
        rc = pvi[1]
        r0 = jnp.maximum(rc - 3, 0)
        r1 = jnp.minimum(rc + 3, _H - 1)
        c0 = jnp.maximum(cc - 3, 0)
        c1 = jnp.minimum(cc + 3, _W - 1)
        ch = s * _QPS + qd
        pair = lax.shift_right_logical(slot, 1)
        par = jnp.bitwise_and(slot, 1)
        for m in range(4):
            k = iota + 16 * m
            a = lax.shift_right_logical(k * 9363, 16)
            kb = k - a * 7
            h = jnp.minimum(r0 + a, r1)
            w = jnp.minimum(c0 + kb, c1)
            mr = jnp.where(h == 0, 4 - rc,
                           jnp.where(h == _H - 1, rc - (_H - 5), 1))
            mc = jnp.where(w == 0, 4 - cc,
                           jnp.where(w == _W - 1, cc - (_W - 5), 1))
            pix = pixbase + h * _W + w
            gidxbuf[b, pl.ds(16 * m, 16)] = pix
            sibuf[pair, pl.ds(par * 64 + 16 * m, 16)] = pix * _NQH + ch
            wts[b, pl.ds(16 * m, 16)] = (mr * mc).astype(jnp.float32)

    def _issue(b):
        pltpu.async_copy(feat.at[gidxbuf.at[b]], rows.at[b], gsems[b])

    def _compute(slot, b):
        qd = jnp.minimum(slot, count - 1)
        qv = [qeown[qd, pl.ds(16 * t, 16)] for t in range(16)]
        for m in range(4):
            def _cell(ci, lv):
                cell = 16 * m + ci
                accs = [rows[b, cell, pl.ds(16 * t, 16)] * qv[t]
                        for t in range(4)]
                for t in range(4, 16):
                    accs[t % 4] = (accs[t % 4]
                                   + rows[b, cell, pl.ds(16 * t, 16)] * qv[t])
                acc = (accs[0] + accs[1]) + (accs[2] + accs[3])
                for shift in (8, 4, 2, 1):
                    red[pl.ds(0, 16)] = acc
                    red[pl.ds(16, 16)] = acc
                    acc = acc + red[pl.ds(shift, 16)]
                return jnp.where(iota == ci, acc, lv)

            lv = lax.fori_loop(0, 16, _cell, jnp.zeros((16,), jnp.float32))
            logbuf[pl.ds(slot * 64 + 16 * m, 16)] = (
                lv * wts[b, pl.ds(16 * m, 16)])

    for b in range(4):
        _build(b, b)
        _issue(b)

    def _group(grp, carry):
        for b in range(4):
            slot = grp * 4 + b
            pltpu.make_async_copy(feat.at[gidxbuf.at[b]], rows.at[b],
                                  gsems[b]).wait()
            _compute(slot, b)

            @pl.when(grp < 4)
            def _():
                _build(slot + 4, b)
                _issue(b)

        @pl.when(grp < 4)
        def _():
            for u in range(2):
                pltpu.async_copy(
                    zbuf,
                    out.at[pl.ds(zbase + (grp * 2 + u) * _ZBUF, _ZBUF)],
                    zsem)
        return carry

    lax.fori_loop(0, 5, _group, 0)

    for j in range(_NZD):
        pltpu.make_async_copy(
            zbuf, out.at[pl.ds(zbase + j * _ZBUF, _ZBUF)], zsem).wait()
    plsc.subcore_barrier()
    sds = [pltpu.async_copy(logbuf.at[pl.ds(128 * j, 128)],
                            out.at[sibuf.at[j]], ssem)
           for j in range(_SLOTS // 2)]
    for d in sds:
        d.wait()


def kernel(stacked_feature_map, queries, query_batch_offsets,
           query_positions, image_spatial_shapes,
           W0, b0, W1, b1, W2, b2, W3, b3):
    qe = pl.pallas_call(
        _mlp_body,
        out_shape=jax.ShapeDtypeStruct((_NQ, _D), jnp.float32),
    )(queries, W0, b0.reshape(1, _D), W1, b1.reshape(1, _D),
      W2, b2.reshape(1, _D), W3, b3.reshape(1, _D))

    feat = stacked_feature_map[..., _L - 1, :].reshape(_B * _HW, _D)
    posf = query_positions.reshape(_NQ * 2)

    sc_fn = pl.kernel(
        _sc_body,
        out_type=jax.ShapeDtypeStruct((_NFLAT,), jnp.float32),
        mesh=plsc.VectorSubcoreMesh(core_axis_name="c", subcore_axis_name="s"),
        scratch_types=[
            pltpu.VMEM((_ZBUF,), jnp.float32),
            pltpu.VMEM((_NQ * 2 + 32,), jnp.float32),
            pltpu.VMEM((32,), jnp.int32),
            pltpu.VMEM((32, _D), jnp.float32),
            pltpu.VMEM((64,), jnp.int32),
            pltpu.VMEM((4, 64), jnp.int32),
            pltpu.VMEM((4, 64), jnp.float32),
            pltpu.VMEM((4, 64, _D), jnp.float32),
            pltpu.VMEM((_SLOTS * 64,), jnp.float32),
            pltpu.VMEM((_SLOTS // 2, 128), jnp.int32),
            pltpu.VMEM((32,), jnp.float32),
            pltpu.SemaphoreType.DMA,
            pltpu.SemaphoreType.DMA,
            pltpu.SemaphoreType.DMA,
            pltpu.SemaphoreType.DMA,
            pltpu.SemaphoreType.DMA,
            pltpu.SemaphoreType.DMA,
        ],
    )
    outf = sc_fn(feat, qe, posf)
    return outf.reshape(_B, _H, _W, _NQH)
